# baseline (speedup 1.0000x reference)
import functools

import jax
import jax.numpy as jnp
from jax import lax
from jax.experimental import pallas as pl
from jax.experimental.pallas import tpu as pltpu
from jax.experimental.pallas import tpu_sc as plsc

N = 10000
NPAD = 10240
NC, NS = 2, 16
RPT = NPAD // NS
CHUNK = 256
GRP = 4
AGRP = 1
D_IN, D_HID, D_OUT = 256, 256, 128
H1 = D_HID // NC
W128 = 128
DEGW = 16
RB = 1000

_mesh = functools.partial(
    plsc.VectorSubcoreMesh, core_axis_name="c", subcore_axis_name="s",
    num_cores=NC, num_subcores=NS)


def _deg_body(ngrp, dst_hbm, ones_hbm, zeros_hbm, deg_out,
              *scr):
  acc = scr[0]
  idxd = list(scr[1:1 + GRP])
  ones_v = scr[1 + GRP]
  isem, ssem = scr[2 + GRP:]
  c = lax.axis_index("c")
  s = lax.axis_index("s")
  pltpu.sync_copy(zeros_hbm, acc.at[pl.ds(s * RPT, RPT)])
  pltpu.sync_copy(ones_hbm, ones_v)
  plsc.subcore_barrier()
  base = (c * NS + s) * ngrp * GRP

  def step(k, carry):
    q = (base + k * GRP) * CHUNK
    ld = [pltpu.async_copy(dst_hbm.at[pl.ds(q + j * CHUNK, CHUNK)],
                           idxd[j], isem) for j in range(GRP)]
    st = []
    for j in range(GRP):
      ld[j].wait()
      st.append(pltpu.async_copy(ones_v, acc.at[idxd[j]], ssem, add=True))
    for d in st:
      d.wait()
    return carry

  lax.fori_loop(0, ngrp, step, 0)
  plsc.subcore_barrier()
  pltpu.sync_copy(acc.at[pl.ds(s * RPT, RPT)],
                  deg_out.at[c, pl.ds(s * RPT, RPT)])


def _deg_kernel(dst_pad, ones, zeros):
  ngrp = dst_pad.shape[0] // (NC * NS * CHUNK * GRP)
  return pl.kernel(
      functools.partial(_deg_body, ngrp),
      out_type=jax.ShapeDtypeStruct((NC, NPAD, W128), jnp.float32),
      mesh=_mesh(),
      scratch_types=(
          [pltpu.VMEM_SHARED((NPAD, W128), jnp.float32)]
          + [pltpu.VMEM((CHUNK,), jnp.int32) for _ in range(GRP)]
          + [pltpu.VMEM((CHUNK, W128), jnp.float32)]
          + [pltpu.SemaphoreType.DMA] * 2),
  )(dst_pad, ones, zeros)


def _agg_pipeline(ngrp, base, src_off, g_hbm, src_hbm, dst_hbm,
                  acc, idxs, idxd, rows, isem, gsem, ssem):
  grp = len(rows)

  def scatter_wait(j):
    pltpu.make_async_copy(rows[j], acc.at[idxd[j]], ssem).wait()

  def step(k, carry):
    q = (base + k * grp) * CHUNK
    ld = []
    for j in range(grp):
      @pl.when(k > 0)
      def _(j=j):
        scatter_wait(j)
      ld.append(pltpu.async_copy(
          src_hbm.at[pl.ds(src_off + q + j * CHUNK, CHUNK)], idxs[j], isem))
      ld.append(pltpu.async_copy(
          dst_hbm.at[pl.ds(q + j * CHUNK, CHUNK)], idxd[j], isem))
    gt = []
    for j in range(grp):
      ld[2 * j].wait()
      ld[2 * j + 1].wait()
      gt.append(pltpu.async_copy(g_hbm.at[idxs[j]], rows[j], gsem))
    for j in range(grp):
      gt[j].wait()
      pltpu.async_copy(rows[j], acc.at[idxd[j]], ssem, add=True)
    return carry

  lax.fori_loop(0, ngrp, step, 0)
  for j in range(grp):
    scatter_wait(j)


def _agg_body(ngrp, epad, gcat_hbm, srcq_hbm, dst_hbm, zeros_hbm,
              out, *scr):
  acc = scr[0]
  idxs = list(scr[1:1 + AGRP])
  idxd = list(scr[1 + AGRP:1 + 2 * AGRP])
  rows = list(scr[1 + 2 * AGRP:1 + 3 * AGRP])
  isem, gsem, ssem = scr[1 + 3 * AGRP:]
  c = lax.axis_index("c")
  s = lax.axis_index("s")
  pltpu.sync_copy(zeros_hbm, acc.at[pl.ds(s * RPT, RPT)])
  plsc.subcore_barrier()
  _agg_pipeline(ngrp, s * ngrp * AGRP, c * epad, gcat_hbm, srcq_hbm, dst_hbm,
                acc, idxs, idxd, rows, isem, gsem, ssem)
  plsc.subcore_barrier()
  pltpu.sync_copy(acc.at[pl.ds(s * RPT, RPT)],
                  out.at[c, pl.ds(s * RPT, RPT)])


def _agg_scratch(hw):
  return ([pltpu.VMEM_SHARED((NPAD, hw), jnp.float32)]
          + [pltpu.VMEM((CHUNK,), jnp.int32) for _ in range(2 * AGRP)]
          + [pltpu.VMEM((CHUNK, hw), jnp.float32) for _ in range(AGRP)]
          + [pltpu.SemaphoreType.DMA] * 3)


def _agg_kernel(gcat, srcq, dst_pad, zeros):
  epad = dst_pad.shape[0]
  ngrp = epad // (NS * CHUNK * AGRP)
  return pl.kernel(
      functools.partial(_agg_body, ngrp, epad),
      out_type=jax.ShapeDtypeStruct((NC, NPAD, H1), jnp.float32),
      mesh=_mesh(),
      scratch_types=_agg_scratch(H1),
  )(gcat, srcq, dst_pad, zeros)


def _agg2_body(ngrp, g_hbm, src_hbm, dst_hbm, zeros_hbm,
               out, *scr):
  acc = scr[0]
  idxs = list(scr[1:1 + AGRP])
  idxd = list(scr[1 + AGRP:1 + 2 * AGRP])
  rows = list(scr[1 + 2 * AGRP:1 + 3 * AGRP])
  isem, gsem, ssem = scr[1 + 3 * AGRP:]
  c = lax.axis_index("c")
  s = lax.axis_index("s")
  pltpu.sync_copy(zeros_hbm, acc.at[pl.ds(s * RPT, RPT)])
  plsc.subcore_barrier()
  _agg_pipeline(ngrp, (c * NS + s) * ngrp * AGRP, 0, g_hbm, src_hbm, dst_hbm,
                acc, idxs, idxd, rows, isem, gsem, ssem)
  plsc.subcore_barrier()
  pltpu.sync_copy(acc.at[pl.ds(s * RPT, RPT)],
                  out.at[c, pl.ds(s * RPT, RPT)])


def _agg2_kernel(g, src_pad, dst_pad, zeros):
  ngrp = src_pad.shape[0] // (NC * NS * CHUNK * AGRP)
  return pl.kernel(
      functools.partial(_agg2_body, ngrp),
      out_type=jax.ShapeDtypeStruct((NC, NPAD, D_OUT), jnp.float32),
      mesh=_mesh(),
      scratch_types=_agg_scratch(D_OUT),
  )(g, src_pad, dst_pad, zeros)


def _dis(dega, degb):
  deg = dega[:, 0] + degb[:, 0] + 1.0
  return lax.rsqrt(deg)


def _mm1_body(x_ref, w_ref, dega_ref, degb_ref, gc_ref):
  h = jnp.dot(x_ref[...], w_ref[...], preferred_element_type=jnp.float32)
  g = h * _dis(dega_ref[...], degb_ref[...])[:, None]
  gc_ref[0] = g[:, :H1]
  gc_ref[1] = g[:, H1:]


def _mm1(x, w1, dega, degb):
  return pl.pallas_call(
      _mm1_body,
      grid=(N // RB,),
      in_specs=[
          pl.BlockSpec((RB, D_IN), lambda i: (i, 0)),
          pl.BlockSpec((D_IN, D_HID), lambda i: (0, 0)),
          pl.BlockSpec((RB, DEGW), lambda i: (i, 0)),
          pl.BlockSpec((RB, DEGW), lambda i: (i, 0)),
      ],
      out_specs=pl.BlockSpec((2, RB, H1), lambda i: (0, i, 0)),
      out_shape=jax.ShapeDtypeStruct((2, N, H1), jnp.float32),
  )(x, w1, dega, degb)


def _mm2_body(a1a_ref, a1b_ref, gc_ref, dega_ref, degb_ref,
              w2_ref, b1_ref, g2_ref):
  dis = _dis(dega_ref[...], degb_ref[...])[:, None]
  z0 = jnp.maximum(dis * (a1a_ref[...] + gc_ref[0]) + b1_ref[0, :H1], 0.0)
  z1 = jnp.maximum(dis * (a1b_ref[...] + gc_ref[1]) + b1_ref[0, H1:], 0.0)
  z = jnp.concatenate([z0, z1], axis=1)
  g2_ref[...] = jnp.dot(z, w2_ref[...], preferred_element_type=jnp.float32) * dis


def _mm2(a1a, a1b, gc, dega, degb, w2, b1):
  return pl.pallas_call(
      _mm2_body,
      grid=(N // RB,),
      in_specs=[
          pl.BlockSpec((RB, H1), lambda i: (i, 0)),
          pl.BlockSpec((RB, H1), lambda i: (i, 0)),
          pl.BlockSpec((2, RB, H1), lambda i: (0, i, 0)),
          pl.BlockSpec((RB, DEGW), lambda i: (i, 0)),
          pl.BlockSpec((RB, DEGW), lambda i: (i, 0)),
          pl.BlockSpec((D_HID, D_OUT), lambda i: (0, 0)),
          pl.BlockSpec((1, D_HID), lambda i: (0, 0)),
      ],
      out_specs=pl.BlockSpec((RB, D_OUT), lambda i: (i, 0)),
      out_shape=jax.ShapeDtypeStruct((N, D_OUT), jnp.float32),
  )(a1a, a1b, gc, dega, degb, w2, b1)


def _fin_body(a2a_ref, a2b_ref, g2_ref, dega_ref, degb_ref,
              b2_ref, out_ref):
  dis = _dis(dega_ref[...], degb_ref[...])[:, None]
  out_ref[...] = (dis * (a2a_ref[...] + a2b_ref[...] + g2_ref[...])
                  + b2_ref[0, :])


def _fin(a2a, a2b, g2, dega, degb, b2):
  return pl.pallas_call(
      _fin_body,
      grid=(N // RB,),
      in_specs=[
          pl.BlockSpec((RB, D_OUT), lambda i: (i, 0)),
          pl.BlockSpec((RB, D_OUT), lambda i: (i, 0)),
          pl.BlockSpec((RB, D_OUT), lambda i: (i, 0)),
          pl.BlockSpec((RB, DEGW), lambda i: (i, 0)),
          pl.BlockSpec((RB, DEGW), lambda i: (i, 0)),
          pl.BlockSpec((1, D_OUT), lambda i: (0, 0)),
      ],
      out_specs=pl.BlockSpec((RB, D_OUT), lambda i: (i, 0)),
      out_shape=jax.ShapeDtypeStruct((N, D_OUT), jnp.float32),
  )(a2a, a2b, g2, dega, degb, b2)


def kernel(x, edge_index, W1, b1, W2, b2):
  e = edge_index.shape[1]
  gran = NC * NS * CHUNK * GRP
  epad = ((e + gran - 1) // gran) * gran
  src = jnp.concatenate(
      [edge_index[0], jnp.zeros((epad - e,), jnp.int32)])
  dst = jnp.concatenate(
      [edge_index[1], jnp.full((epad - e,), N, jnp.int32)])
  srcq = jnp.concatenate([src, src + N])

  ones = jnp.ones((CHUNK, W128), jnp.float32)
  zeros128 = jnp.zeros((RPT, W128), jnp.float32)
  zeros_h1 = jnp.zeros((RPT, H1), jnp.float32)
  zeros_h2 = jnp.zeros((RPT, D_OUT), jnp.float32)

  deg2 = _deg_kernel(dst, ones, zeros128)
  dega, degb = deg2[0, :N, :DEGW], deg2[1, :N, :DEGW]
  gc = _mm1(x, W1, dega, degb)
  gcat = gc.reshape(2 * N, H1)
  a1 = _agg_kernel(gcat, srcq, dst, zeros_h1)
  g2 = _mm2(a1[0, :N], a1[1, :N], gc, dega, degb, W2, b1.reshape(1, D_HID))
  a2 = _agg2_kernel(g2, src, dst, zeros_h2)
  return _fin(a2[0, :N], a2[1, :N], g2, dega, degb, b2.reshape(1, D_OUT))


# speedup vs baseline: 1.0546x; 1.0546x over previous
import functools

import jax
import jax.numpy as jnp
from jax import lax
from jax.experimental import pallas as pl
from jax.experimental.pallas import tpu as pltpu
from jax.experimental.pallas import tpu_sc as plsc

N = 10000
NPAD = 10240
NC, NS = 2, 16
RPT = NPAD // NS
CHUNK = 160
GRP = 4
AGRP = 2
D_IN, D_HID, D_OUT = 256, 256, 128
H1 = D_HID // NC
W128 = 128
DEGW = 16
RB = 1000

_mesh = functools.partial(
    plsc.VectorSubcoreMesh, core_axis_name="c", subcore_axis_name="s",
    num_cores=NC, num_subcores=NS)


def _deg_body(ngrp, dst_hbm, ones_hbm, zeros_hbm, deg_out,
              *scr):
  acc = scr[0]
  idxd = list(scr[1:1 + GRP])
  ones_v = scr[1 + GRP]
  isem, ssem = scr[2 + GRP:]
  c = lax.axis_index("c")
  s = lax.axis_index("s")
  pltpu.sync_copy(zeros_hbm, acc.at[pl.ds(s * RPT, RPT)])
  pltpu.sync_copy(ones_hbm, ones_v)
  plsc.subcore_barrier()
  base = (c * NS + s) * ngrp * GRP

  def step(k, carry):
    q = (base + k * GRP) * CHUNK
    ld = [pltpu.async_copy(dst_hbm.at[pl.ds(q + j * CHUNK, CHUNK)],
                           idxd[j], isem) for j in range(GRP)]
    st = []
    for j in range(GRP):
      ld[j].wait()
      st.append(pltpu.async_copy(ones_v, acc.at[idxd[j]], ssem, add=True))
    for d in st:
      d.wait()
    return carry

  lax.fori_loop(0, ngrp, step, 0)
  plsc.subcore_barrier()
  pltpu.sync_copy(acc.at[pl.ds(s * RPT, RPT)],
                  deg_out.at[c, pl.ds(s * RPT, RPT)])


def _deg_kernel(dst_pad, ones, zeros):
  ngrp = dst_pad.shape[0] // (NC * NS * CHUNK * GRP)
  return pl.kernel(
      functools.partial(_deg_body, ngrp),
      out_type=jax.ShapeDtypeStruct((NC, NPAD, W128), jnp.float32),
      mesh=_mesh(),
      scratch_types=(
          [pltpu.VMEM_SHARED((NPAD, W128), jnp.float32)]
          + [pltpu.VMEM((CHUNK,), jnp.int32) for _ in range(GRP)]
          + [pltpu.VMEM((CHUNK, W128), jnp.float32)]
          + [pltpu.SemaphoreType.DMA] * 2),
  )(dst_pad, ones, zeros)


def _agg_pipeline(ngrp, base, src_off, g_hbm, src_hbm, dst_hbm,
                  acc, idxs, idxd, rows, isem, gsem, ssem):
  grp = len(rows)

  def scatter_wait(j):
    pltpu.make_async_copy(rows[j], acc.at[idxd[j]], ssem).wait()

  def step(k, carry):
    q = (base + k * grp) * CHUNK
    ld = []
    for j in range(grp):
      @pl.when(k > 0)
      def _(j=j):
        scatter_wait(j)
      ld.append(pltpu.async_copy(
          src_hbm.at[pl.ds(src_off + q + j * CHUNK, CHUNK)], idxs[j], isem))
      ld.append(pltpu.async_copy(
          dst_hbm.at[pl.ds(q + j * CHUNK, CHUNK)], idxd[j], isem))
    gt = []
    for j in range(grp):
      ld[2 * j].wait()
      ld[2 * j + 1].wait()
      gt.append(pltpu.async_copy(g_hbm.at[idxs[j]], rows[j], gsem))
    for j in range(grp):
      gt[j].wait()
      pltpu.async_copy(rows[j], acc.at[idxd[j]], ssem, add=True)
    return carry

  lax.fori_loop(0, ngrp, step, 0)
  for j in range(grp):
    scatter_wait(j)


def _agg_body(ngrp, epad, gcat_hbm, srcq_hbm, dst_hbm, zeros_hbm,
              out, *scr):
  acc = scr[0]
  idxs = list(scr[1:1 + AGRP])
  idxd = list(scr[1 + AGRP:1 + 2 * AGRP])
  rows = list(scr[1 + 2 * AGRP:1 + 3 * AGRP])
  isem, gsem, ssem = scr[1 + 3 * AGRP:]
  c = lax.axis_index("c")
  s = lax.axis_index("s")
  pltpu.sync_copy(zeros_hbm, acc.at[pl.ds(s * RPT, RPT)])
  plsc.subcore_barrier()
  _agg_pipeline(ngrp, s * ngrp * AGRP, c * epad, gcat_hbm, srcq_hbm, dst_hbm,
                acc, idxs, idxd, rows, isem, gsem, ssem)
  plsc.subcore_barrier()
  pltpu.sync_copy(acc.at[pl.ds(s * RPT, RPT)],
                  out.at[c, pl.ds(s * RPT, RPT)])


def _agg_scratch(hw):
  return ([pltpu.VMEM_SHARED((NPAD, hw), jnp.float32)]
          + [pltpu.VMEM((CHUNK,), jnp.int32) for _ in range(2 * AGRP)]
          + [pltpu.VMEM((CHUNK, hw), jnp.float32) for _ in range(AGRP)]
          + [pltpu.SemaphoreType.DMA] * 3)


def _agg_kernel(gcat, srcq, dst_pad, zeros):
  epad = dst_pad.shape[0]
  ngrp = epad // (NS * CHUNK * AGRP)
  return pl.kernel(
      functools.partial(_agg_body, ngrp, epad),
      out_type=jax.ShapeDtypeStruct((NC, NPAD, H1), jnp.float32),
      mesh=_mesh(),
      scratch_types=_agg_scratch(H1),
  )(gcat, srcq, dst_pad, zeros)


def _agg2_body(ngrp, g_hbm, src_hbm, dst_hbm, zeros_hbm,
               out, *scr):
  acc = scr[0]
  idxs = list(scr[1:1 + AGRP])
  idxd = list(scr[1 + AGRP:1 + 2 * AGRP])
  rows = list(scr[1 + 2 * AGRP:1 + 3 * AGRP])
  isem, gsem, ssem = scr[1 + 3 * AGRP:]
  c = lax.axis_index("c")
  s = lax.axis_index("s")
  pltpu.sync_copy(zeros_hbm, acc.at[pl.ds(s * RPT, RPT)])
  plsc.subcore_barrier()
  _agg_pipeline(ngrp, (c * NS + s) * ngrp * AGRP, 0, g_hbm, src_hbm, dst_hbm,
                acc, idxs, idxd, rows, isem, gsem, ssem)
  plsc.subcore_barrier()
  pltpu.sync_copy(acc.at[pl.ds(s * RPT, RPT)],
                  out.at[c, pl.ds(s * RPT, RPT)])


def _agg2_kernel(g, src_pad, dst_pad, zeros):
  ngrp = src_pad.shape[0] // (NC * NS * CHUNK * AGRP)
  return pl.kernel(
      functools.partial(_agg2_body, ngrp),
      out_type=jax.ShapeDtypeStruct((NC, NPAD, D_OUT), jnp.float32),
      mesh=_mesh(),
      scratch_types=_agg_scratch(D_OUT),
  )(g, src_pad, dst_pad, zeros)


def _dis(dega, degb):
  deg = dega[:, 0] + degb[:, 0] + 1.0
  return lax.rsqrt(deg)


def _mm1_body(x_ref, w_ref, dega_ref, degb_ref, gc_ref):
  h = jnp.dot(x_ref[...], w_ref[...], preferred_element_type=jnp.float32)
  g = h * _dis(dega_ref[...], degb_ref[...])[:, None]
  gc_ref[0] = g[:, :H1]
  gc_ref[1] = g[:, H1:]


def _mm1(x, w1, dega, degb):
  return pl.pallas_call(
      _mm1_body,
      grid=(N // RB,),
      in_specs=[
          pl.BlockSpec((RB, D_IN), lambda i: (i, 0)),
          pl.BlockSpec((D_IN, D_HID), lambda i: (0, 0)),
          pl.BlockSpec((RB, DEGW), lambda i: (i, 0)),
          pl.BlockSpec((RB, DEGW), lambda i: (i, 0)),
      ],
      out_specs=pl.BlockSpec((2, RB, H1), lambda i: (0, i, 0)),
      out_shape=jax.ShapeDtypeStruct((2, N, H1), jnp.float32),
  )(x, w1, dega, degb)


def _mm2_body(a1a_ref, a1b_ref, gc_ref, dega_ref, degb_ref,
              w2_ref, b1_ref, g2_ref):
  dis = _dis(dega_ref[...], degb_ref[...])[:, None]
  z0 = jnp.maximum(dis * (a1a_ref[...] + gc_ref[0]) + b1_ref[0, :H1], 0.0)
  z1 = jnp.maximum(dis * (a1b_ref[...] + gc_ref[1]) + b1_ref[0, H1:], 0.0)
  z = jnp.concatenate([z0, z1], axis=1)
  g2_ref[...] = jnp.dot(z, w2_ref[...], preferred_element_type=jnp.float32) * dis


def _mm2(a1a, a1b, gc, dega, degb, w2, b1):
  return pl.pallas_call(
      _mm2_body,
      grid=(N // RB,),
      in_specs=[
          pl.BlockSpec((RB, H1), lambda i: (i, 0)),
          pl.BlockSpec((RB, H1), lambda i: (i, 0)),
          pl.BlockSpec((2, RB, H1), lambda i: (0, i, 0)),
          pl.BlockSpec((RB, DEGW), lambda i: (i, 0)),
          pl.BlockSpec((RB, DEGW), lambda i: (i, 0)),
          pl.BlockSpec((D_HID, D_OUT), lambda i: (0, 0)),
          pl.BlockSpec((1, D_HID), lambda i: (0, 0)),
      ],
      out_specs=pl.BlockSpec((RB, D_OUT), lambda i: (i, 0)),
      out_shape=jax.ShapeDtypeStruct((N, D_OUT), jnp.float32),
  )(a1a, a1b, gc, dega, degb, w2, b1)


def _fin_body(a2a_ref, a2b_ref, g2_ref, dega_ref, degb_ref,
              b2_ref, out_ref):
  dis = _dis(dega_ref[...], degb_ref[...])[:, None]
  out_ref[...] = (dis * (a2a_ref[...] + a2b_ref[...] + g2_ref[...])
                  + b2_ref[0, :])


def _fin(a2a, a2b, g2, dega, degb, b2):
  return pl.pallas_call(
      _fin_body,
      grid=(N // RB,),
      in_specs=[
          pl.BlockSpec((RB, D_OUT), lambda i: (i, 0)),
          pl.BlockSpec((RB, D_OUT), lambda i: (i, 0)),
          pl.BlockSpec((RB, D_OUT), lambda i: (i, 0)),
          pl.BlockSpec((RB, DEGW), lambda i: (i, 0)),
          pl.BlockSpec((RB, DEGW), lambda i: (i, 0)),
          pl.BlockSpec((1, D_OUT), lambda i: (0, 0)),
      ],
      out_specs=pl.BlockSpec((RB, D_OUT), lambda i: (i, 0)),
      out_shape=jax.ShapeDtypeStruct((N, D_OUT), jnp.float32),
  )(a2a, a2b, g2, dega, degb, b2)


def kernel(x, edge_index, W1, b1, W2, b2):
  e = edge_index.shape[1]
  gran = NC * NS * CHUNK * GRP
  epad = ((e + gran - 1) // gran) * gran
  src = jnp.concatenate(
      [edge_index[0], jnp.zeros((epad - e,), jnp.int32)])
  dst = jnp.concatenate(
      [edge_index[1], jnp.full((epad - e,), N, jnp.int32)])
  srcq = jnp.concatenate([src, src + N])

  ones = jnp.ones((CHUNK, W128), jnp.float32)
  zeros128 = jnp.zeros((RPT, W128), jnp.float32)
  zeros_h1 = jnp.zeros((RPT, H1), jnp.float32)
  zeros_h2 = jnp.zeros((RPT, D_OUT), jnp.float32)

  deg2 = _deg_kernel(dst, ones, zeros128)
  dega, degb = deg2[0, :N, :DEGW], deg2[1, :N, :DEGW]
  gc = _mm1(x, W1, dega, degb)
  gcat = gc.reshape(2 * N, H1)
  a1 = _agg_kernel(gcat, srcq, dst, zeros_h1)
  g2 = _mm2(a1[0, :N], a1[1, :N], gc, dega, degb, W2, b1.reshape(1, D_HID))
  a2 = _agg2_kernel(g2, src, dst, zeros_h2)
  return _fin(a2[0, :N], a2[1, :N], g2, dega, degb, b2.reshape(1, D_OUT))


# speedup vs baseline: 1.2048x; 1.1425x over previous
import functools

import jax
import jax.numpy as jnp
from jax import lax
from jax.experimental import pallas as pl
from jax.experimental.pallas import tpu as pltpu
from jax.experimental.pallas import tpu_sc as plsc

N = 10000
NPAD = 10240
NC, NS = 2, 16
RPT = NPAD // NS
CHUNK = 128
GRP = 4
AGRP = 2
D_IN, D_HID, D_OUT = 256, 256, 128
H1 = D_HID // NC
W128 = 128
DEGW = 16
RB = 1000

_mesh = functools.partial(
    plsc.VectorSubcoreMesh, core_axis_name="c", subcore_axis_name="s",
    num_cores=NC, num_subcores=NS)


def _deg_body(ngrp, dst_hbm, ones_hbm, zeros_hbm, deg_out,
              *scr):
  acc = scr[0]
  idxd = list(scr[1:1 + GRP])
  ones_v = scr[1 + GRP]
  isem, ssem = scr[2 + GRP:]
  c = lax.axis_index("c")
  s = lax.axis_index("s")
  pltpu.sync_copy(zeros_hbm, acc.at[pl.ds(s * RPT, RPT)])
  pltpu.sync_copy(ones_hbm, ones_v)
  plsc.subcore_barrier()
  base = (c * NS + s) * ngrp * GRP

  def step(k, carry):
    q = (base + k * GRP) * CHUNK
    ld = [pltpu.async_copy(dst_hbm.at[pl.ds(q + j * CHUNK, CHUNK)],
                           idxd[j], isem) for j in range(GRP)]
    st = []
    for j in range(GRP):
      ld[j].wait()
      st.append(pltpu.async_copy(ones_v, acc.at[idxd[j]], ssem, add=True))
    for d in st:
      d.wait()
    return carry

  lax.fori_loop(0, ngrp, step, 0)
  plsc.subcore_barrier()
  pltpu.sync_copy(acc.at[pl.ds(s * RPT, RPT)],
                  deg_out.at[c, pl.ds(s * RPT, RPT)])


def _deg_kernel(dst_pad, ones, zeros):
  ngrp = dst_pad.shape[0] // (NC * NS * CHUNK * GRP)
  return pl.kernel(
      functools.partial(_deg_body, ngrp),
      out_type=jax.ShapeDtypeStruct((NC, NPAD, W128), jnp.float32),
      mesh=_mesh(),
      scratch_types=(
          [pltpu.VMEM_SHARED((NPAD, W128), jnp.float32)]
          + [pltpu.VMEM((CHUNK,), jnp.int32) for _ in range(GRP)]
          + [pltpu.VMEM((CHUNK, W128), jnp.float32)]
          + [pltpu.SemaphoreType.DMA] * 2),
  )(dst_pad, ones, zeros)


def _agg_pipeline(nchunks, base, src_off, g_hbm, src_hbm, dst_hbm,
                  acc, idxs, idxd, rows, isem, gsem, ssem):
  def idx_issue(g, p):
    q = (base + g) * CHUNK
    pltpu.async_copy(src_hbm.at[pl.ds(src_off + q, CHUNK)], idxs[p], isem)
    pltpu.async_copy(dst_hbm.at[pl.ds(q, CHUNK)], idxd[p], isem)

  def idx_wait(p):
    pltpu.make_async_copy(src_hbm.at[pl.ds(0, CHUNK)], idxs[p], isem).wait()
    pltpu.make_async_copy(dst_hbm.at[pl.ds(0, CHUNK)], idxd[p], isem).wait()

  def gather_issue(p):
    pltpu.async_copy(g_hbm.at[idxs[p]], rows[p], gsem)

  def gather_wait(p):
    pltpu.make_async_copy(g_hbm.at[idxs[p]], rows[p], gsem).wait()

  def scatter_issue(p):
    pltpu.async_copy(rows[p], acc.at[idxd[p]], ssem, add=True)

  def scatter_wait(p):
    pltpu.make_async_copy(rows[p], acc.at[idxd[p]], ssem).wait()

  nit = nchunks // 2
  idx_issue(0, 0)
  idx_wait(0)
  gather_issue(0)

  def step(k, carry):
    for j in (0, 1):
      g = 2 * k + j
      p = j
      q = 1 - j
      if j == 0:
        @pl.when(k > 0)
        def _():
          scatter_wait(1)
        idx_issue(g + 1, q)
      else:
        scatter_wait(0)

        @pl.when(k < nit - 1)
        def _():
          idx_issue(g + 1, q)
      gather_wait(p)
      scatter_issue(p)
      if j == 0:
        idx_wait(q)
        gather_issue(q)
      else:
        @pl.when(k < nit - 1)
        def _():
          idx_wait(q)
          gather_issue(q)
    return carry

  lax.fori_loop(0, nit, step, 0)
  scatter_wait(1)


def _agg_body(nchunks, epad, gcat_hbm, srcq_hbm, dst_hbm, zeros_hbm,
              out, *scr):
  acc = scr[0]
  idxs = list(scr[1:1 + AGRP])
  idxd = list(scr[1 + AGRP:1 + 2 * AGRP])
  rows = list(scr[1 + 2 * AGRP:1 + 3 * AGRP])
  isem, gsem, ssem = scr[1 + 3 * AGRP:]
  c = lax.axis_index("c")
  s = lax.axis_index("s")
  pltpu.sync_copy(zeros_hbm, acc.at[pl.ds(s * RPT, RPT)])
  plsc.subcore_barrier()
  _agg_pipeline(nchunks, s * nchunks, c * epad, gcat_hbm, srcq_hbm, dst_hbm,
                acc, idxs, idxd, rows, isem, gsem, ssem)
  plsc.subcore_barrier()
  pltpu.sync_copy(acc.at[pl.ds(s * RPT, RPT)],
                  out.at[c, pl.ds(s * RPT, RPT)])


def _agg_scratch(hw):
  return ([pltpu.VMEM_SHARED((NPAD, hw), jnp.float32)]
          + [pltpu.VMEM((CHUNK,), jnp.int32) for _ in range(2 * AGRP)]
          + [pltpu.VMEM((CHUNK, hw), jnp.float32) for _ in range(AGRP)]
          + [pltpu.SemaphoreType.DMA] * 3)


def _agg_kernel(gcat, srcq, dst_pad, zeros):
  epad = dst_pad.shape[0]
  nchunks = epad // (NS * CHUNK)
  return pl.kernel(
      functools.partial(_agg_body, nchunks, epad),
      out_type=jax.ShapeDtypeStruct((NC, NPAD, H1), jnp.float32),
      mesh=_mesh(),
      scratch_types=_agg_scratch(H1),
  )(gcat, srcq, dst_pad, zeros)


def _agg2_body(nchunks, g_hbm, src_hbm, dst_hbm, zeros_hbm,
               out, *scr):
  acc = scr[0]
  idxs = list(scr[1:1 + AGRP])
  idxd = list(scr[1 + AGRP:1 + 2 * AGRP])
  rows = list(scr[1 + 2 * AGRP:1 + 3 * AGRP])
  isem, gsem, ssem = scr[1 + 3 * AGRP:]
  c = lax.axis_index("c")
  s = lax.axis_index("s")
  pltpu.sync_copy(zeros_hbm, acc.at[pl.ds(s * RPT, RPT)])
  plsc.subcore_barrier()
  _agg_pipeline(nchunks, (c * NS + s) * nchunks, 0, g_hbm, src_hbm, dst_hbm,
                acc, idxs, idxd, rows, isem, gsem, ssem)
  plsc.subcore_barrier()
  pltpu.sync_copy(acc.at[pl.ds(s * RPT, RPT)],
                  out.at[c, pl.ds(s * RPT, RPT)])


def _agg2_kernel(g, src_pad, dst_pad, zeros):
  nchunks = src_pad.shape[0] // (NC * NS * CHUNK)
  return pl.kernel(
      functools.partial(_agg2_body, nchunks),
      out_type=jax.ShapeDtypeStruct((NC, NPAD, D_OUT), jnp.float32),
      mesh=_mesh(),
      scratch_types=_agg_scratch(D_OUT),
  )(g, src_pad, dst_pad, zeros)


def _dis(dega, degb):
  deg = dega[:, 0] + degb[:, 0] + 1.0
  return lax.rsqrt(deg)


def _mm1_body(x_ref, w_ref, dega_ref, degb_ref, gc_ref):
  h = jnp.dot(x_ref[...], w_ref[...], preferred_element_type=jnp.float32)
  g = h * _dis(dega_ref[...], degb_ref[...])[:, None]
  gc_ref[0] = g[:, :H1]
  gc_ref[1] = g[:, H1:]


def _mm1(x, w1, dega, degb):
  return pl.pallas_call(
      _mm1_body,
      grid=(N // RB,),
      in_specs=[
          pl.BlockSpec((RB, D_IN), lambda i: (i, 0)),
          pl.BlockSpec((D_IN, D_HID), lambda i: (0, 0)),
          pl.BlockSpec((RB, DEGW), lambda i: (i, 0)),
          pl.BlockSpec((RB, DEGW), lambda i: (i, 0)),
      ],
      out_specs=pl.BlockSpec((2, RB, H1), lambda i: (0, i, 0)),
      out_shape=jax.ShapeDtypeStruct((2, N, H1), jnp.float32),
  )(x, w1, dega, degb)


def _mm2_body(a1a_ref, a1b_ref, gc_ref, dega_ref, degb_ref,
              w2_ref, b1_ref, g2_ref):
  dis = _dis(dega_ref[...], degb_ref[...])[:, None]
  z0 = jnp.maximum(dis * (a1a_ref[...] + gc_ref[0]) + b1_ref[0, :H1], 0.0)
  z1 = jnp.maximum(dis * (a1b_ref[...] + gc_ref[1]) + b1_ref[0, H1:], 0.0)
  z = jnp.concatenate([z0, z1], axis=1)
  g2_ref[...] = jnp.dot(z, w2_ref[...], preferred_element_type=jnp.float32) * dis


def _mm2(a1a, a1b, gc, dega, degb, w2, b1):
  return pl.pallas_call(
      _mm2_body,
      grid=(N // RB,),
      in_specs=[
          pl.BlockSpec((RB, H1), lambda i: (i, 0)),
          pl.BlockSpec((RB, H1), lambda i: (i, 0)),
          pl.BlockSpec((2, RB, H1), lambda i: (0, i, 0)),
          pl.BlockSpec((RB, DEGW), lambda i: (i, 0)),
          pl.BlockSpec((RB, DEGW), lambda i: (i, 0)),
          pl.BlockSpec((D_HID, D_OUT), lambda i: (0, 0)),
          pl.BlockSpec((1, D_HID), lambda i: (0, 0)),
      ],
      out_specs=pl.BlockSpec((RB, D_OUT), lambda i: (i, 0)),
      out_shape=jax.ShapeDtypeStruct((N, D_OUT), jnp.float32),
  )(a1a, a1b, gc, dega, degb, w2, b1)


def _fin_body(a2a_ref, a2b_ref, g2_ref, dega_ref, degb_ref,
              b2_ref, out_ref):
  dis = _dis(dega_ref[...], degb_ref[...])[:, None]
  out_ref[...] = (dis * (a2a_ref[...] + a2b_ref[...] + g2_ref[...])
                  + b2_ref[0, :])


def _fin(a2a, a2b, g2, dega, degb, b2):
  return pl.pallas_call(
      _fin_body,
      grid=(N // RB,),
      in_specs=[
          pl.BlockSpec((RB, D_OUT), lambda i: (i, 0)),
          pl.BlockSpec((RB, D_OUT), lambda i: (i, 0)),
          pl.BlockSpec((RB, D_OUT), lambda i: (i, 0)),
          pl.BlockSpec((RB, DEGW), lambda i: (i, 0)),
          pl.BlockSpec((RB, DEGW), lambda i: (i, 0)),
          pl.BlockSpec((1, D_OUT), lambda i: (0, 0)),
      ],
      out_specs=pl.BlockSpec((RB, D_OUT), lambda i: (i, 0)),
      out_shape=jax.ShapeDtypeStruct((N, D_OUT), jnp.float32),
  )(a2a, a2b, g2, dega, degb, b2)


def kernel(x, edge_index, W1, b1, W2, b2):
  e = edge_index.shape[1]
  gran = NC * NS * CHUNK * GRP
  epad = ((e + gran - 1) // gran) * gran
  src = jnp.concatenate(
      [edge_index[0], jnp.zeros((epad - e,), jnp.int32)])
  dst = jnp.concatenate(
      [edge_index[1], jnp.full((epad - e,), N, jnp.int32)])
  srcq = jnp.concatenate([src, src + N])

  ones = jnp.ones((CHUNK, W128), jnp.float32)
  zeros128 = jnp.zeros((RPT, W128), jnp.float32)
  zeros_h1 = jnp.zeros((RPT, H1), jnp.float32)
  zeros_h2 = jnp.zeros((RPT, D_OUT), jnp.float32)

  deg2 = _deg_kernel(dst, ones, zeros128)
  dega, degb = deg2[0, :N, :DEGW], deg2[1, :N, :DEGW]
  gc = _mm1(x, W1, dega, degb)
  gcat = gc.reshape(2 * N, H1)
  a1 = _agg_kernel(gcat, srcq, dst, zeros_h1)
  g2 = _mm2(a1[0, :N], a1[1, :N], gc, dega, degb, W2, b1.reshape(1, D_HID))
  a2 = _agg2_kernel(g2, src, dst, zeros_h2)
  return _fin(a2[0, :N], a2[1, :N], g2, dega, degb, b2.reshape(1, D_OUT))
